# baseline (speedup 1.0000x reference)
import functools

import numpy as np
import jax
import jax.numpy as jnp
from jax import lax
from jax.experimental import pallas as pl
from jax.experimental.pallas import tpu as pltpu
from jax.experimental.pallas import tpu_sc as plsc

H = 768
E = 3
FF = 4 * H
HK = H // 2

N = 32768
RT = 2048
BLK = 1024
NBE = N // BLK
NSLOTS = NBE + E
FFC = FF // 2

NW = 32
PER_W = N // NW
CH = 128
NCH = PER_W // CH

BLK_SHIFT = BLK.bit_length() - 1


def _router_kernel(x_ref, w1_ref, b1_ref, w2_ref, b2_ref, tril_ref,
                   dst_ref, bs_ref, be_ref, run_ref):
    i = pl.program_id(0)

    @pl.when(i == 0)
    def _init():
        run_ref[...] = jnp.zeros_like(run_ref)

    x = x_ref[...]
    h = jnp.maximum(
        jnp.dot(x, w1_ref[...], preferred_element_type=jnp.float32)
        + b1_ref[...], 0.0)
    logits = (jnp.dot(h, w2_ref[...], preferred_element_type=jnp.float32)
              + b2_ref[...])
    idx = jnp.argmax(logits, axis=-1, keepdims=True)
    eiota = lax.broadcasted_iota(jnp.int32, (RT, E), 1)
    onehot = (idx == eiota).astype(jnp.float32)
    within = jnp.dot(tril_ref[...], onehot,
                     preferred_element_type=jnp.float32)
    rank = within + run_ref[...]
    base = eiota.astype(jnp.float32) * float(N)
    dstf = jnp.sum(onehot * (base + rank), axis=1, keepdims=True)
    dst_ref[...] = dstf.astype(jnp.int32)
    new_run = run_ref[...] + jnp.sum(onehot, axis=0, keepdims=True)
    run_ref[...] = new_run

    @pl.when(i == N // RT - 1)
    def _tables():
        cnt = new_run.astype(jnp.int32)
        u = (cnt + (BLK - 1)) >> BLK_SHIFT
        u0 = u[:, 0:1]
        u01 = u0 + u[:, 1:2]
        k = lax.broadcasted_iota(jnp.int32, (1, NSLOTS), 1)
        ex = ((k >= u0).astype(jnp.int32) + (k >= u01).astype(jnp.int32))
        start = jnp.where(ex == 0, 0, jnp.where(ex == 1, u0, u01))
        within_b = jnp.clip(k - start, 0, NBE - 1)
        bs_ref[...] = ex * NBE + within_b
        be_ref[...] = ex


def _router(x2d, W1, b1, W2, b2, tril):
    return pl.pallas_call(
        _router_kernel,
        grid=(N // RT,),
        in_specs=[
            pl.BlockSpec((RT, H), lambda i: (i, 0)),
            pl.BlockSpec((H, HK), lambda i: (0, 0)),
            pl.BlockSpec((1, HK), lambda i: (0, 0)),
            pl.BlockSpec((HK, E), lambda i: (0, 0)),
            pl.BlockSpec((1, E), lambda i: (0, 0)),
            pl.BlockSpec((RT, RT), lambda i: (0, 0)),
        ],
        out_specs=[
            pl.BlockSpec((RT, 1), lambda i: (i, 0)),
            pl.BlockSpec((1, NSLOTS), lambda i: (0, 0)),
            pl.BlockSpec((1, NSLOTS), lambda i: (0, 0)),
        ],
        out_shape=[
            jax.ShapeDtypeStruct((N, 1), jnp.int32),
            jax.ShapeDtypeStruct((1, NSLOTS), jnp.int32),
            jax.ShapeDtypeStruct((1, NSLOTS), jnp.int32),
        ],
        scratch_shapes=[pltpu.VMEM((1, E), jnp.float32)],
        compiler_params=pltpu.CompilerParams(
            dimension_semantics=("arbitrary",),
        ),
    )(x2d, W1, b1, W2, b2, tril)


_SC_MESH = plsc.VectorSubcoreMesh(core_axis_name="c", subcore_axis_name="s")


@functools.partial(
    pl.kernel, mesh=_SC_MESH,
    out_type=jax.ShapeDtypeStruct((E * N, H), jnp.float32),
    scratch_types=[
        pltpu.VMEM((CH,), jnp.int32),
        pltpu.VMEM((CH, H), jnp.float32),
        pltpu.SemaphoreType.DMA,
    ],
)
def _sc_scatter(x_hbm, dst_hbm, xs_hbm, idx_v, rows_v, sem):
    wid = lax.axis_index("s") * 2 + lax.axis_index("c")
    for c in range(NCH):
        base = wid * PER_W + c * CH
        pltpu.sync_copy(dst_hbm.at[pl.ds(base, CH)], idx_v)
        pltpu.sync_copy(x_hbm.at[pl.ds(base, CH)], rows_v)
        pltpu.async_copy(rows_v, xs_hbm.at[idx_v], sem).wait()


@functools.partial(
    pl.kernel, mesh=_SC_MESH,
    out_type=jax.ShapeDtypeStruct((N, H), jnp.float32),
    scratch_types=[
        pltpu.VMEM((CH,), jnp.int32),
        pltpu.VMEM((CH, H), jnp.float32),
        pltpu.SemaphoreType.DMA,
    ],
)
def _sc_gather(ys_hbm, dst_hbm, out_hbm, idx_v, rows_v, sem):
    wid = lax.axis_index("s") * 2 + lax.axis_index("c")
    for c in range(NCH):
        base = wid * PER_W + c * CH
        pltpu.sync_copy(dst_hbm.at[pl.ds(base, CH)], idx_v)
        pltpu.async_copy(ys_hbm.at[idx_v], rows_v, sem).wait()
        pltpu.sync_copy(rows_v, out_hbm.at[pl.ds(base, CH)])


def _ffn_kernel(bs_ref, be_ref, xs_ref, wa_ref, ba_ref, wb_ref, bb_ref,
                ys_ref):
    del bs_ref, be_ref
    c = pl.program_id(1)
    x = xs_ref[...]
    eh = jnp.dot(x, wa_ref[0], preferred_element_type=jnp.float32)
    eh = eh + ba_ref[0]
    eh = 0.5 * eh * (1.0 + lax.erf(eh * 0.7071067811865476))
    part = jnp.dot(eh, wb_ref[0], preferred_element_type=jnp.float32)

    @pl.when(c == 0)
    def _first():
        ys_ref[...] = part + bb_ref[0] + x

    @pl.when(c != 0)
    def _rest():
        ys_ref[...] = ys_ref[...] + part


def _ffn(bs, be, xs, Wa, ba, Wb, bb):
    grid_spec = pltpu.PrefetchScalarGridSpec(
        num_scalar_prefetch=2,
        grid=(NSLOTS, FF // FFC),
        in_specs=[
            pl.BlockSpec((BLK, H), lambda j, c, bs, be: (bs[j], 0)),
            pl.BlockSpec((1, H, FFC), lambda j, c, bs, be: (be[j], 0, c)),
            pl.BlockSpec((1, 1, FFC), lambda j, c, bs, be: (be[j], 0, c)),
            pl.BlockSpec((1, FFC, H), lambda j, c, bs, be: (be[j], c, 0)),
            pl.BlockSpec((1, 1, H), lambda j, c, bs, be: (be[j], 0, 0)),
        ],
        out_specs=pl.BlockSpec((BLK, H), lambda j, c, bs, be: (bs[j], 0)),
    )
    return pl.pallas_call(
        _ffn_kernel,
        grid_spec=grid_spec,
        out_shape=jax.ShapeDtypeStruct((E * N, H), jnp.float32),
        compiler_params=pltpu.CompilerParams(
            dimension_semantics=("arbitrary", "arbitrary"),
        ),
    )(bs, be, xs, Wa, ba, Wb, bb)


_TRIL = np.tril(np.ones((RT, RT), np.float32), -1)


@jax.jit
def _moe(x2d, W1, b1, W2, b2, Wa, ba, Wb, bb):
    dst2d, bs2d, be2d = _router(x2d, W1, b1, W2, b2, _TRIL)
    dst = dst2d.reshape(N)
    xs = _sc_scatter(x2d, dst)
    ys = _ffn(bs2d.reshape(NSLOTS), be2d.reshape(NSLOTS), xs, Wa, ba, Wb, bb)
    return _sc_gather(ys, dst)


def kernel(hidden_states, W1, b1, W2, b2, Wa, ba, Wb, bb):
    B, S, _ = hidden_states.shape
    x2d = hidden_states.reshape(B * S, H)
    out = _moe(x2d, W1, b1.reshape(1, HK), W2, b2.reshape(1, E),
               Wa, ba.reshape(E, 1, FF), Wb, bb.reshape(E, 1, H))
    return out.reshape(B, S, H)


# speedup vs baseline: 1.0728x; 1.0728x over previous
import functools

import numpy as np
import jax
import jax.numpy as jnp
from jax import lax
from jax.experimental import pallas as pl
from jax.experimental.pallas import tpu as pltpu
from jax.experimental.pallas import tpu_sc as plsc

H = 768
E = 3
FF = 4 * H
HK = H // 2

N = 32768
RT = 1024
BLK = 512
NBE = N // BLK
NSLOTS = NBE + E

NW = 32
PER_W = N // NW
CH = 128
NCH = PER_W // CH

BLK_SHIFT = BLK.bit_length() - 1


def _router_kernel(x_ref, w1_ref, b1_ref, w2_ref, b2_ref, tril_ref,
                   dst_ref, bs_ref, be_ref, run_ref):
    i = pl.program_id(0)

    @pl.when(i == 0)
    def _init():
        run_ref[...] = jnp.zeros_like(run_ref)

    x = x_ref[...]
    h = jnp.maximum(
        jnp.dot(x, w1_ref[...], preferred_element_type=jnp.float32)
        + b1_ref[...], 0.0)
    logits = (jnp.dot(h, w2_ref[...], preferred_element_type=jnp.float32)
              + b2_ref[...])
    idx = jnp.argmax(logits, axis=-1, keepdims=True)
    eiota = lax.broadcasted_iota(jnp.int32, (RT, E), 1)
    onehot = (idx == eiota).astype(jnp.float32)
    within = jnp.dot(tril_ref[...], onehot,
                     preferred_element_type=jnp.float32)
    rank = within + run_ref[...]
    base = eiota.astype(jnp.float32) * float(N)
    dstf = jnp.sum(onehot * (base + rank), axis=1, keepdims=True)
    dst_ref[...] = dstf.astype(jnp.int32)
    new_run = run_ref[...] + jnp.sum(onehot, axis=0, keepdims=True)
    run_ref[...] = new_run

    @pl.when(i == N // RT - 1)
    def _tables():
        cnt = new_run.astype(jnp.int32)
        u = (cnt + (BLK - 1)) >> BLK_SHIFT
        u0 = u[:, 0:1]
        u01 = u0 + u[:, 1:2]
        k = lax.broadcasted_iota(jnp.int32, (1, NSLOTS), 1)
        ex = ((k >= u0).astype(jnp.int32) + (k >= u01).astype(jnp.int32))
        start = jnp.where(ex == 0, 0, jnp.where(ex == 1, u0, u01))
        within_b = jnp.clip(k - start, 0, NBE - 1)
        bs_ref[...] = ex * NBE + within_b
        be_ref[...] = ex


def _router(x2d, W1, b1, W2, b2, tril):
    return pl.pallas_call(
        _router_kernel,
        grid=(N // RT,),
        in_specs=[
            pl.BlockSpec((RT, H), lambda i: (i, 0)),
            pl.BlockSpec((H, HK), lambda i: (0, 0)),
            pl.BlockSpec((1, HK), lambda i: (0, 0)),
            pl.BlockSpec((HK, E), lambda i: (0, 0)),
            pl.BlockSpec((1, E), lambda i: (0, 0)),
            pl.BlockSpec((RT, RT), lambda i: (0, 0)),
        ],
        out_specs=[
            pl.BlockSpec((RT, 1), lambda i: (i, 0)),
            pl.BlockSpec((1, NSLOTS), lambda i: (0, 0)),
            pl.BlockSpec((1, NSLOTS), lambda i: (0, 0)),
        ],
        out_shape=[
            jax.ShapeDtypeStruct((N, 1), jnp.int32),
            jax.ShapeDtypeStruct((1, NSLOTS), jnp.int32),
            jax.ShapeDtypeStruct((1, NSLOTS), jnp.int32),
        ],
        scratch_shapes=[pltpu.VMEM((1, E), jnp.float32)],
        compiler_params=pltpu.CompilerParams(
            dimension_semantics=("arbitrary",),
        ),
    )(x2d, W1, b1, W2, b2, tril)


_SC_MESH = plsc.VectorSubcoreMesh(core_axis_name="c", subcore_axis_name="s")


@functools.partial(
    pl.kernel, mesh=_SC_MESH,
    out_type=jax.ShapeDtypeStruct((E * N, H), jnp.float32),
    scratch_types=[
        pltpu.VMEM((CH,), jnp.int32),
        pltpu.VMEM((CH, H), jnp.float32),
        pltpu.SemaphoreType.DMA,
    ],
)
def _sc_scatter(x_hbm, dst_hbm, xs_hbm, idx_v, rows_v, sem):
    wid = lax.axis_index("s") * 2 + lax.axis_index("c")
    for c in range(NCH):
        base = wid * PER_W + c * CH
        pltpu.sync_copy(dst_hbm.at[pl.ds(base, CH)], idx_v)
        pltpu.sync_copy(x_hbm.at[pl.ds(base, CH)], rows_v)
        pltpu.async_copy(rows_v, xs_hbm.at[idx_v], sem).wait()


@functools.partial(
    pl.kernel, mesh=_SC_MESH,
    out_type=jax.ShapeDtypeStruct((N, H), jnp.float32),
    scratch_types=[
        pltpu.VMEM((CH,), jnp.int32),
        pltpu.VMEM((CH, H), jnp.float32),
        pltpu.SemaphoreType.DMA,
    ],
)
def _sc_gather(ys_hbm, dst_hbm, out_hbm, idx_v, rows_v, sem):
    wid = lax.axis_index("s") * 2 + lax.axis_index("c")
    for c in range(NCH):
        base = wid * PER_W + c * CH
        pltpu.sync_copy(dst_hbm.at[pl.ds(base, CH)], idx_v)
        pltpu.async_copy(ys_hbm.at[idx_v], rows_v, sem).wait()
        pltpu.sync_copy(rows_v, out_hbm.at[pl.ds(base, CH)])


def _ffn_kernel(bs_ref, be_ref, xs_ref, wa_ref, ba_ref, wb_ref, bb_ref,
                ys_ref):
    del bs_ref, be_ref
    x = xs_ref[...]
    eh = jnp.dot(x, wa_ref[0], preferred_element_type=jnp.float32)
    eh = eh + ba_ref[0]
    eh = 0.5 * eh * (1.0 + lax.erf(eh * 0.7071067811865476))
    ys = jnp.dot(eh, wb_ref[0], preferred_element_type=jnp.float32)
    ys_ref[...] = ys + bb_ref[0] + x


def _ffn(bs, be, xs, Wa, ba, Wb, bb):
    grid_spec = pltpu.PrefetchScalarGridSpec(
        num_scalar_prefetch=2,
        grid=(NSLOTS,),
        in_specs=[
            pl.BlockSpec((BLK, H), lambda j, bs, be: (bs[j], 0)),
            pl.BlockSpec((1, H, FF), lambda j, bs, be: (be[j], 0, 0)),
            pl.BlockSpec((1, 1, FF), lambda j, bs, be: (be[j], 0, 0)),
            pl.BlockSpec((1, FF, H), lambda j, bs, be: (be[j], 0, 0)),
            pl.BlockSpec((1, 1, H), lambda j, bs, be: (be[j], 0, 0)),
        ],
        out_specs=pl.BlockSpec((BLK, H), lambda j, bs, be: (bs[j], 0)),
    )
    return pl.pallas_call(
        _ffn_kernel,
        grid_spec=grid_spec,
        out_shape=jax.ShapeDtypeStruct((E * N, H), jnp.float32),
        compiler_params=pltpu.CompilerParams(
            dimension_semantics=("arbitrary",),
        ),
    )(bs, be, xs, Wa, ba, Wb, bb)


_TRIL = np.tril(np.ones((RT, RT), np.float32), -1)


@jax.jit
def _moe(x2d, W1, b1, W2, b2, Wa, ba, Wb, bb):
    dst2d, bs2d, be2d = _router(x2d, W1, b1, W2, b2, _TRIL)
    dst = dst2d.reshape(N)
    xs = _sc_scatter(x2d, dst)
    ys = _ffn(bs2d.reshape(NSLOTS), be2d.reshape(NSLOTS), xs, Wa, ba, Wb, bb)
    return _sc_gather(ys, dst)


def kernel(hidden_states, W1, b1, W2, b2, Wa, ba, Wb, bb):
    B, S, _ = hidden_states.shape
    x2d = hidden_states.reshape(B * S, H)
    out = _moe(x2d, W1, b1.reshape(1, HK), W2, b2.reshape(1, E),
               Wa, ba.reshape(E, 1, FF), Wb, bb.reshape(E, 1, H))
    return out.reshape(B, S, H)
